# baseline (speedup 1.0000x reference)
import functools

import jax
import jax.numpy as jnp
from jax import lax
from jax.experimental import pallas as pl
from jax.experimental.pallas import tpu as pltpu
from jax.experimental.pallas import tpu_sc as plsc

N = 50000
E = 800000
H = 64

NPAD = 50176
ROWS_PER_SUB = NPAD // 16
ZCH = 196
CHUNK = 128
NCHUNKS = E // CHUNK
ITERS_PER_SUB = -(-NCHUNKS // 16)
ITERS_PER_WORKER = -(-NCHUNKS // 32)

_SC_MESH = plsc.VectorSubcoreMesh(core_axis_name="c", subcore_axis_name="s")
_SC_PARAMS = pltpu.CompilerParams(use_tc_tiling_on_sc=False)
_f32 = jnp.float32


def _zero_stripe(sub, zbuf, table, width):
    def zrow(j, carry):
        for k in range(0, width, 16):
            zbuf[j, pl.ds(k, 16)] = jnp.zeros((16,), _f32)
        return carry
    lax.fori_loop(0, CHUNK, zrow, None)
    row0 = sub * ROWS_PER_SUB
    nfull, rem = divmod(ROWS_PER_SUB, CHUNK)
    for k in range(nfull):
        pltpu.sync_copy(zbuf, table.at[pl.ds(row0 + k * CHUNK, CHUNK)])
    if rem:
        pltpu.sync_copy(zbuf.at[pl.ds(0, rem)],
                        table.at[pl.ds(row0 + nfull * CHUNK, rem)])


def _cnt_body(worker, sub, dst_hbm, out_hbm, idx_d, obuf, c_sp):
    _zero_stripe(sub, obuf, c_sp, 16)

    def fill_ones(j, carry):
        obuf[j, pl.ds(0, 16)] = jnp.ones((16,), _f32)
        return carry
    lax.fori_loop(0, CHUNK, fill_ones, None)
    plsc.subcore_barrier()

    def chunk_body(i, carry):
        ch = worker + i * 32

        @pl.when(ch < NCHUNKS)
        def _():
            pltpu.sync_copy(dst_hbm.at[pl.ds(ch * CHUNK, CHUNK)], idx_d)
            pltpu.sync_copy(obuf, c_sp.at[idx_d], add=True)
        return carry
    lax.fori_loop(0, ITERS_PER_WORKER, chunk_body, None)
    plsc.subcore_barrier()
    row0 = sub * ROWS_PER_SUB
    pltpu.sync_copy(c_sp.at[pl.ds(row0, ROWS_PER_SUB)],
                    out_hbm.at[pl.ds(row0, ROWS_PER_SUB)])


@functools.partial(
    pl.kernel,
    out_type=(jax.ShapeDtypeStruct((NPAD, 16), _f32),
              jax.ShapeDtypeStruct((NPAD, 16), _f32)),
    mesh=_SC_MESH,
    scratch_types=[
        pltpu.VMEM((CHUNK,), jnp.int32),
        pltpu.VMEM((CHUNK, 16), _f32),
        pltpu.VMEM_SHARED((NPAD, 16), _f32),
    ],
    compiler_params=_SC_PARAMS,
)
def _sc_count(dst_hbm, cnt0_hbm, cnt1_hbm, idx_d, obuf, c_sp):
    c = lax.axis_index("c")
    s = lax.axis_index("s")
    worker = s * 2 + c

    @pl.when(c == 0)
    def _():
        _cnt_body(worker, s, dst_hbm, cnt0_hbm, idx_d, obuf, c_sp)

    @pl.when(c == 1)
    def _():
        _cnt_body(worker, s, dst_hbm, cnt1_hbm, idx_d, obuf, c_sp)


NSLOT = 3
PAIRS = 130
TAIL_T = 390
CPB = BE_CHUNKS = 25


def _edge_body(sub, src_hbm, dst_hbm, p_hbm, q_hbm, ee_hbm, out_hbm,
               idxs, idxd, ebs, qbs, rbuf, s_sp,
               sem_i, sem_b, sem_g):
    _zero_stripe(sub, rbuf, s_sp, 32)
    plsc.subcore_barrier()

    def chunk_of(t):
        return sub + t * 16

    def issue_front(slot, ch):
        @pl.when(ch < NCHUNKS)
        def _():
            base = ch * CHUNK
            pltpu.async_copy(src_hbm.at[pl.ds(base, CHUNK)], idxs[slot],
                             sem_i[slot])
            pltpu.async_copy(dst_hbm.at[pl.ds(base, CHUNK)], idxd[slot],
                             sem_i[slot])
            pltpu.async_copy(ee_hbm.at[pl.ds(base, CHUNK)], ebs[slot],
                             sem_b[slot])

    def issue_gather(slot, ch):
        @pl.when(ch < NCHUNKS)
        def _():
            pltpu.make_async_copy(src_hbm.at[pl.ds(0, CHUNK)], idxs[slot],
                                  sem_i[slot]).wait()
            pltpu.make_async_copy(dst_hbm.at[pl.ds(0, CHUNK)], idxd[slot],
                                  sem_i[slot]).wait()
            pltpu.make_async_copy(ee_hbm.at[pl.ds(0, CHUNK)], ebs[slot],
                                  sem_b[slot]).wait()
            pltpu.async_copy(p_hbm.at[idxs[slot]], ebs[slot], sem_g[slot],
                             add=True)
            pltpu.async_copy(q_hbm.at[idxd[slot]], qbs[slot], sem_g[slot])

    def do_compute(slot, ch):
        @pl.when(ch < NCHUNKS)
        def _():
            pltpu.make_async_copy(p_hbm.at[idxs[slot]], ebs[slot],
                                  sem_g[slot]).wait()
            pltpu.make_async_copy(q_hbm.at[idxd[slot]], qbs[slot],
                                  sem_g[slot]).wait()

            def comp(j, carry2):
                for k in (0, 16):
                    v = ebs[slot][j, pl.ds(k, 16)] + qbs[slot][j, pl.ds(k, 16)]
                    rbuf[j, pl.ds(k, 16)] = jnp.maximum(v, 0.0)
                return carry2
            lax.fori_loop(0, CHUNK, comp, None)
            pltpu.sync_copy(rbuf, s_sp.at[idxd[slot]], add=True)

    issue_front(0, chunk_of(0))
    issue_front(1, chunk_of(1))
    issue_gather(0, chunk_of(0))

    def triple(i, carry):
        t0 = 3 * i
        for d in range(3):
            issue_front((d + 2) % NSLOT, chunk_of(t0 + d + 2))
            issue_gather((d + 1) % NSLOT, chunk_of(t0 + d + 1))
            do_compute(d, chunk_of(t0 + d))
        return carry
    lax.fori_loop(0, PAIRS, triple, None)
    do_compute(TAIL_T % NSLOT, chunk_of(TAIL_T))

    plsc.subcore_barrier()
    row0 = sub * ROWS_PER_SUB
    pltpu.sync_copy(s_sp.at[pl.ds(row0, ROWS_PER_SUB)],
                    out_hbm.at[pl.ds(row0, ROWS_PER_SUB)])


@functools.partial(
    pl.kernel,
    out_type=(jax.ShapeDtypeStruct((NPAD, 32), _f32),
              jax.ShapeDtypeStruct((NPAD, 32), _f32)),
    mesh=_SC_MESH,
    scratch_types=(
        [pltpu.VMEM((CHUNK,), jnp.int32)] * 6
        + [pltpu.VMEM((CHUNK, 32), _f32)] * 6
        + [pltpu.VMEM((CHUNK, 32), _f32),
           pltpu.VMEM_SHARED((NPAD, 32), _f32)]
        + [pltpu.SemaphoreType.DMA] * 9
    ),
    compiler_params=_SC_PARAMS,
)
def _sc_edge_pass(src_hbm, dst_hbm, p0, p1, q0, q1, e0, e1, s0_out, s1_out,
                  is0, is1, is2, id0, id1, id2,
                  eb0, eb1, eb2, qb0, qb1, qb2,
                  rbuf, s_sp,
                  si0, si1, si2, sb0, sb1, sb2, sg0, sg1, sg2):
    c = lax.axis_index("c")
    s = lax.axis_index("s")
    idxs = (is0, is1, is2)
    idxd = (id0, id1, id2)
    ebs = (eb0, eb1, eb2)
    qbs = (qb0, qb1, qb2)
    sem_i = (si0, si1, si2)
    sem_b = (sb0, sb1, sb2)
    sem_g = (sg0, sg1, sg2)

    @pl.when(c == 0)
    def _():
        _edge_body(s, src_hbm, dst_hbm, p0, q0, e0, s0_out,
                   idxs, idxd, ebs, qbs, rbuf, s_sp,
                   sem_i, sem_b, sem_g)

    @pl.when(c == 1)
    def _():
        _edge_body(s, src_hbm, dst_hbm, p1, q1, e1, s1_out,
                   idxs, idxd, ebs, qbs, rbuf, s_sp,
                   sem_i, sem_b, sem_g)


BN = 512
GRID_N = NPAD // BN
BE = 3200
GRID_E = E // BE


def _dot(a, b):
    return jnp.dot(a, b, preferred_element_type=_f32)


def _pack(v):
    v3 = jnp.reshape(v, (v.shape[0] // 4, 4, 32))
    return jnp.concatenate([v3[:, u, :] for u in range(4)], axis=1)


def _unpack(sb):
    parts = [jnp.reshape(sb[:, 32 * u:32 * (u + 1)], (sb.shape[0], 1, 32))
             for u in range(4)]
    return jnp.reshape(jnp.concatenate(parts, axis=1), (4 * sb.shape[0], 32))


def _node_tc(x_ref, nw1, nb1, nw2, nb2, wj0, wj1, wi0, wi1,
             h_out, p0, p1, q0, q1):
    z = jnp.maximum(_dot(x_ref[...], nw1[...]) + nb1[...], 0.0)
    h = _dot(z, nw2[...]) + nb2[...]
    h_out[...] = h
    p0[...] = _pack(_dot(h, wj0[...]))
    p1[...] = _pack(_dot(h, wj1[...]))
    q0[...] = _pack(_dot(h, wi0[...]))
    q1[...] = _pack(_dot(h, wi1[...]))


def _edgefeat_tc(ea_ref, ew1, eb1, ew2, eb2, we0, we1, we2, mb0, mb1, mb2,
                 o00, o01, o10, o11, o20, o21):
    z = jnp.maximum(_dot(ea_ref[...], ew1[...]) + eb1[...], 0.0)
    e = _dot(z, ew2[...]) + eb2[...]
    for t, (oa, ob) in (
            (_dot(e, we0[...]) + mb0[...], (o00, o01)),
            (_dot(e, we1[...]) + mb1[...], (o10, o11)),
            (_dot(e, we2[...]) + mb2[...], (o20, o21)),
    ):
        oa[...] = _pack(t[:, 0:32])
        ob[...] = _pack(t[:, 32:64])


def _make_update_tc(with_pq):
    def body(h_ref, s0, s1, c0, c1, mw2, mb2, uw1h, uw1a, ub1, uw2, ub2,
             *rest):
        if with_pq:
            wj0, wj1, wi0, wi1, h_out, p0, p1, q0, q1 = rest
        else:
            h_out, = rest
        hb = h_ref[...]
        s = jnp.concatenate([_unpack(s0[...]), _unpack(s1[...])], axis=1)
        cnt_raw = c0[...][:, 0:1] + c1[...][:, 0:1]
        cnt = jnp.maximum(cnt_raw, 1.0)
        has_edges = jnp.minimum(cnt_raw, 1.0)
        aggr = _dot(s, mw2[...]) / cnt + has_edges * mb2[...]
        z = jnp.maximum(_dot(hb, uw1h[...]) + _dot(aggr, uw1a[...])
                        + ub1[...], 0.0)
        hn = hb + _dot(z, uw2[...]) + ub2[...]
        h_out[...] = hn
        if with_pq:
            p0[...] = _pack(_dot(hn, wj0[...]))
            p1[...] = _pack(_dot(hn, wj1[...]))
            q0[...] = _pack(_dot(hn, wi0[...]))
            q1[...] = _pack(_dot(hn, wi1[...]))
    return body


def _head_tc(h_ref, w1, b1, w2p, b2p, y_out):
    z = jnp.maximum(_dot(h_ref[...], w1[...]) + b1[...], 0.0)
    y_out[...] = _dot(z, w2p[...]) + b2p[...]


def _full(shape):
    return pl.BlockSpec(shape, lambda i: (0,) * len(shape))


def _rows(width):
    return pl.BlockSpec((BN, width), lambda i: (i, 0))


def _erows(width):
    return pl.BlockSpec((BE, width), lambda i: (i, 0))


def kernel(x, edge_index, edge_attr, node_W1, node_b1, node_W2, node_b2,
           edge_W1, edge_b1, edge_W2, edge_b2,
           msg_W1, msg_b1, msg_W2, msg_b2,
           upd_W1, upd_b1, upd_W2, upd_b2,
           head_W1, head_b1, head_W2, head_b2):
    x_p = jnp.zeros((NPAD, 8), _f32).at[:N, :5].set(x)
    src = edge_index[0]
    dst = edge_index[1]


    nW1p = jnp.zeros((8, H), _f32).at[:5].set(node_W1)
    r1 = lambda b: b.reshape(1, -1)

    mWj = msg_W1[:, 0:H, :]
    mWi = msg_W1[:, H:2 * H, :]
    mWe = msg_W1[:, 2 * H:3 * H, :]

    pq_pack_specs = [pl.BlockSpec((BN // 4, 128), lambda i: (i, 0))] * 4
    pq_pack_shapes = [jax.ShapeDtypeStruct((NPAD // 4, 128), _f32)] * 4

    node_call = pl.pallas_call(
        _node_tc,
        grid=(GRID_N,),
        in_specs=[_rows(8), _full((8, H)), _full((1, H)), _full((H, H)),
                  _full((1, H)), _full((H, 32)), _full((H, 32)),
                  _full((H, 32)), _full((H, 32))],
        out_specs=[_rows(H)] + pq_pack_specs,
        out_shape=[jax.ShapeDtypeStruct((NPAD, H), _f32)] + pq_pack_shapes,
    )
    h, P0, P1, Q0, Q1 = node_call(
        x_p, nW1p, r1(node_b1), node_W2, r1(node_b2),
        mWj[0][:, 0:32], mWj[0][:, 32:64], mWi[0][:, 0:32], mWi[0][:, 32:64])

    edgefeat_call = pl.pallas_call(
        _edgefeat_tc,
        grid=(GRID_E,),
        in_specs=[_erows(7), _full((7, H)), _full((1, H)), _full((H, H)),
                  _full((1, H))] + [_full((H, H))] * 3 + [_full((1, H))] * 3,
        out_specs=[pl.BlockSpec((BE // 4, 128), lambda i: (i, 0))] * 6,
        out_shape=[jax.ShapeDtypeStruct((E // 4, 128), _f32)] * 6,
    )
    ee = edgefeat_call(edge_attr, edge_W1, r1(edge_b1), edge_W2, r1(edge_b2),
                       mWe[0], mWe[1], mWe[2],
                       r1(msg_b1[0]), r1(msg_b1[1]), r1(msg_b1[2]))
    ee = [jnp.reshape(a, (E, 32)) for a in ee]

    c0, c1 = _sc_count(dst)

    s_pack_spec = pl.BlockSpec((BN // 4, 128), lambda i: (i, 0))
    upd_in_specs = [_rows(H), s_pack_spec, s_pack_spec, _rows(16), _rows(16),
                    _full((H, H)), _full((1, H)), _full((H, H)),
                    _full((H, H)), _full((1, H)), _full((H, H)),
                    _full((1, H))]
    upd_pq_call = pl.pallas_call(
        _make_update_tc(True),
        grid=(GRID_N,),
        in_specs=upd_in_specs + [_full((H, 32))] * 4,
        out_specs=[_rows(H)] + pq_pack_specs,
        out_shape=[jax.ShapeDtypeStruct((NPAD, H), _f32)] + pq_pack_shapes,
    )
    upd_call = pl.pallas_call(
        _make_update_tc(False),
        grid=(GRID_N,),
        in_specs=upd_in_specs,
        out_specs=[_rows(H)],
        out_shape=[jax.ShapeDtypeStruct((NPAD, H), _f32)],
    )

    for l in range(3):
        s0, s1 = _sc_edge_pass(src, dst,
                               jnp.reshape(P0, (NPAD, 32)),
                               jnp.reshape(P1, (NPAD, 32)),
                               jnp.reshape(Q0, (NPAD, 32)),
                               jnp.reshape(Q1, (NPAD, 32)),
                               ee[2 * l], ee[2 * l + 1])
        common = (h, jnp.reshape(s0, (NPAD // 4, 128)),
                  jnp.reshape(s1, (NPAD // 4, 128)),
                  c0, c1, msg_W2[l], r1(msg_b2[l]),
                  upd_W1[l][0:H], upd_W1[l][H:2 * H], r1(upd_b1[l]),
                  upd_W2[l], r1(upd_b2[l]))
        if l < 2:
            h, P0, P1, Q0, Q1 = upd_pq_call(
                *common,
                mWj[l + 1][:, 0:32], mWj[l + 1][:, 32:64],
                mWi[l + 1][:, 0:32], mWi[l + 1][:, 32:64])
        else:
            h, = upd_call(*common)

    hW2p = jnp.zeros((H, 128), _f32).at[:, 0:2].set(head_W2)
    hb2p = jnp.zeros((1, 128), _f32).at[0, 0:2].set(head_b2)
    head_call = pl.pallas_call(
        _head_tc,
        grid=(1,),
        in_specs=[pl.BlockSpec((8, H), lambda i: (0, 0)), _full((H, H)),
                  _full((1, H)), _full((H, 128)), _full((1, 128))],
        out_specs=pl.BlockSpec((8, 128), lambda i: (0, 0)),
        out_shape=jax.ShapeDtypeStruct((8, 128), _f32),
    )
    y = head_call(h, head_W1, r1(head_b1), hW2p, hb2p)
    return y[0:1, 0:2]


# speedup vs baseline: 1.2921x; 1.2921x over previous
import functools

import jax
import jax.numpy as jnp
from jax import lax
from jax.experimental import pallas as pl
from jax.experimental.pallas import tpu as pltpu
from jax.experimental.pallas import tpu_sc as plsc

N = 50000
E = 800000
H = 64

NPAD = 50176
ROWS_PER_SUB = NPAD // 16
ZCH = 196
CHUNK = 128
NCHUNKS = E // CHUNK
ITERS_PER_SUB = -(-NCHUNKS // 16)
ITERS_PER_WORKER = -(-NCHUNKS // 32)

_SC_MESH = plsc.VectorSubcoreMesh(core_axis_name="c", subcore_axis_name="s")
_SC_PARAMS = pltpu.CompilerParams(use_tc_tiling_on_sc=False)
_f32 = jnp.float32


def _zero_stripe(sub, zbuf, table, width):
    def zrow(j, carry):
        for k in range(0, width, 16):
            zbuf[j, pl.ds(k, 16)] = jnp.zeros((16,), _f32)
        return carry
    lax.fori_loop(0, CHUNK, zrow, None)
    row0 = sub * ROWS_PER_SUB
    nfull, rem = divmod(ROWS_PER_SUB, CHUNK)
    for k in range(nfull):
        pltpu.sync_copy(zbuf, table.at[pl.ds(row0 + k * CHUNK, CHUNK)])
    if rem:
        pltpu.sync_copy(zbuf.at[pl.ds(0, rem)],
                        table.at[pl.ds(row0 + nfull * CHUNK, rem)])


def _cnt_body(worker, sub, dst_hbm, out_hbm, idx_d, obuf, c_sp):
    _zero_stripe(sub, obuf, c_sp, 16)

    def fill_ones(j, carry):
        obuf[j, pl.ds(0, 16)] = jnp.ones((16,), _f32)
        return carry
    lax.fori_loop(0, CHUNK, fill_ones, None)
    plsc.subcore_barrier()

    def chunk_body(i, carry):
        ch = worker + i * 32

        @pl.when(ch < NCHUNKS)
        def _():
            pltpu.sync_copy(dst_hbm.at[pl.ds(ch * CHUNK, CHUNK)], idx_d)
            pltpu.sync_copy(obuf, c_sp.at[idx_d], add=True)
        return carry
    lax.fori_loop(0, ITERS_PER_WORKER, chunk_body, None)
    plsc.subcore_barrier()
    row0 = sub * ROWS_PER_SUB
    pltpu.sync_copy(c_sp.at[pl.ds(row0, ROWS_PER_SUB)],
                    out_hbm.at[pl.ds(row0, ROWS_PER_SUB)])


@functools.partial(
    pl.kernel,
    out_type=(jax.ShapeDtypeStruct((NPAD, 16), _f32),
              jax.ShapeDtypeStruct((NPAD, 16), _f32)),
    mesh=_SC_MESH,
    scratch_types=[
        pltpu.VMEM((CHUNK,), jnp.int32),
        pltpu.VMEM((CHUNK, 16), _f32),
        pltpu.VMEM_SHARED((NPAD, 16), _f32),
    ],
    compiler_params=_SC_PARAMS,
)
def _sc_count(dst_hbm, cnt0_hbm, cnt1_hbm, idx_d, obuf, c_sp):
    c = lax.axis_index("c")
    s = lax.axis_index("s")
    worker = s * 2 + c

    @pl.when(c == 0)
    def _():
        _cnt_body(worker, s, dst_hbm, cnt0_hbm, idx_d, obuf, c_sp)

    @pl.when(c == 1)
    def _():
        _cnt_body(worker, s, dst_hbm, cnt1_hbm, idx_d, obuf, c_sp)


NSLOT = 3
PAIRS = 130
TAIL_T = 390
CPB = BE_CHUNKS = 25


def _edge_body(sub, src_hbm, dst_hbm, p_hbm, q_hbm, ee_hbm, out_hbm,
               idxs, idxd, ebs, qbs, rbuf, s_sp,
               sem_i, sem_b, sem_g):
    _zero_stripe(sub, rbuf, s_sp, 32)
    plsc.subcore_barrier()

    def chunk_of(t):
        return sub + t * 16

    def issue_front(slot, ch):
        @pl.when(ch < NCHUNKS)
        def _():
            blk = ch // CPB
            rbase = (ch % CPB) * 32
            for u in range(4):
                off = blk * BE + u * (BE // 4) + rbase
                pltpu.async_copy(src_hbm.at[pl.ds(off, 32)],
                                 idxs[slot].at[pl.ds(u * 32, 32)],
                                 sem_i[slot])
                pltpu.async_copy(dst_hbm.at[pl.ds(off, 32)],
                                 idxd[slot].at[pl.ds(u * 32, 32)],
                                 sem_i[slot])
                pltpu.async_copy(
                    ee_hbm.at[pl.ds(ch * 32, 32), pl.ds(u * 32, 32)],
                    ebs[slot].at[pl.ds(u * 32, 32)], sem_b[slot])

    def issue_gather(slot, ch):
        @pl.when(ch < NCHUNKS)
        def _():
            for _u in range(8):
                pltpu.make_async_copy(src_hbm.at[pl.ds(0, 32)],
                                      idxs[slot].at[pl.ds(0, 32)],
                                      sem_i[slot]).wait()
            for _u in range(4):
                pltpu.make_async_copy(
                    ee_hbm.at[pl.ds(0, 32), pl.ds(0, 32)],
                    ebs[slot].at[pl.ds(0, 32)], sem_b[slot]).wait()
            for idx in (idxs[slot], idxd[slot]):
                def pi_group(g, carry):
                    n = idx[pl.ds(g * 16, 16)]
                    idx[pl.ds(g * 16, 16)] = ((n & -512) + ((n & 127) << 2)
                                              + ((n >> 7) & 3))
                    return carry
                lax.fori_loop(0, 8, pi_group, None)
            pltpu.async_copy(p_hbm.at[idxs[slot]], ebs[slot], sem_g[slot],
                             add=True)
            pltpu.async_copy(q_hbm.at[idxd[slot]], qbs[slot], sem_g[slot])

    def do_compute(slot, ch):
        @pl.when(ch < NCHUNKS)
        def _():
            pltpu.make_async_copy(p_hbm.at[idxs[slot]], ebs[slot],
                                  sem_g[slot]).wait()
            pltpu.make_async_copy(q_hbm.at[idxd[slot]], qbs[slot],
                                  sem_g[slot]).wait()

            def comp(j, carry2):
                for k in (0, 16):
                    v = ebs[slot][j, pl.ds(k, 16)] + qbs[slot][j, pl.ds(k, 16)]
                    rbuf[j, pl.ds(k, 16)] = jnp.maximum(v, 0.0)
                return carry2
            lax.fori_loop(0, CHUNK, comp, None)
            pltpu.sync_copy(rbuf, s_sp.at[idxd[slot]], add=True)

    issue_front(0, chunk_of(0))
    issue_front(1, chunk_of(1))
    issue_gather(0, chunk_of(0))

    def triple(i, carry):
        t0 = 3 * i
        for d in range(3):
            issue_front((d + 2) % NSLOT, chunk_of(t0 + d + 2))
            issue_gather((d + 1) % NSLOT, chunk_of(t0 + d + 1))
            do_compute(d, chunk_of(t0 + d))
        return carry
    lax.fori_loop(0, PAIRS, triple, None)
    do_compute(TAIL_T % NSLOT, chunk_of(TAIL_T))

    plsc.subcore_barrier()
    row0 = sub * ROWS_PER_SUB
    pltpu.sync_copy(s_sp.at[pl.ds(row0, ROWS_PER_SUB)],
                    out_hbm.at[pl.ds(row0, ROWS_PER_SUB)])


@functools.partial(
    pl.kernel,
    out_type=(jax.ShapeDtypeStruct((NPAD, 32), _f32),
              jax.ShapeDtypeStruct((NPAD, 32), _f32)),
    mesh=_SC_MESH,
    scratch_types=(
        [pltpu.VMEM((CHUNK,), jnp.int32)] * 6
        + [pltpu.VMEM((CHUNK, 32), _f32)] * 6
        + [pltpu.VMEM((CHUNK, 32), _f32),
           pltpu.VMEM_SHARED((NPAD, 32), _f32)]
        + [pltpu.SemaphoreType.DMA] * 9
    ),
    compiler_params=_SC_PARAMS,
)
def _sc_edge_pass(src_hbm, dst_hbm, p0, p1, q0, q1, e0, e1, s0_out, s1_out,
                  is0, is1, is2, id0, id1, id2,
                  eb0, eb1, eb2, qb0, qb1, qb2,
                  rbuf, s_sp,
                  si0, si1, si2, sb0, sb1, sb2, sg0, sg1, sg2):
    c = lax.axis_index("c")
    s = lax.axis_index("s")
    idxs = (is0, is1, is2)
    idxd = (id0, id1, id2)
    ebs = (eb0, eb1, eb2)
    qbs = (qb0, qb1, qb2)
    sem_i = (si0, si1, si2)
    sem_b = (sb0, sb1, sb2)
    sem_g = (sg0, sg1, sg2)

    @pl.when(c == 0)
    def _():
        _edge_body(s, src_hbm, dst_hbm, p0, q0, e0, s0_out,
                   idxs, idxd, ebs, qbs, rbuf, s_sp,
                   sem_i, sem_b, sem_g)

    @pl.when(c == 1)
    def _():
        _edge_body(s, src_hbm, dst_hbm, p1, q1, e1, s1_out,
                   idxs, idxd, ebs, qbs, rbuf, s_sp,
                   sem_i, sem_b, sem_g)


BN = 512
GRID_N = NPAD // BN
BE = 3200
GRID_E = E // BE


def _dot(a, b):
    return jnp.dot(a, b, preferred_element_type=_f32)


def _pack(v):
    q = v.shape[0] // 4
    return jnp.concatenate([v[0:q], v[q:2 * q], v[2 * q:3 * q], v[3 * q:]],
                           axis=1)


def _unpack(sb, width=32):
    return jnp.concatenate([sb[:, u * width:(u + 1) * width]
                            for u in range(128 // width)], axis=0)


def _node_tc(x_ref, nw1, nb1, nw2, nb2, wj0, wj1, wi0, wi1,
             h_out, p0, p1, q0, q1):
    z = jnp.maximum(_dot(x_ref[...], nw1[...]) + nb1[...], 0.0)
    h = _dot(z, nw2[...]) + nb2[...]
    h_out[...] = h
    p0[...] = _pack(_dot(h, wj0[...]))
    p1[...] = _pack(_dot(h, wj1[...]))
    q0[...] = _pack(_dot(h, wi0[...]))
    q1[...] = _pack(_dot(h, wi1[...]))


def _edgefeat_tc(ea_ref, ew1, eb1, ew2, eb2, we0, we1, we2, mb0, mb1, mb2,
                 o00, o01, o10, o11, o20, o21):
    z = jnp.maximum(_dot(ea_ref[...], ew1[...]) + eb1[...], 0.0)
    e = _dot(z, ew2[...]) + eb2[...]
    for t, (oa, ob) in (
            (_dot(e, we0[...]) + mb0[...], (o00, o01)),
            (_dot(e, we1[...]) + mb1[...], (o10, o11)),
            (_dot(e, we2[...]) + mb2[...], (o20, o21)),
    ):
        oa[...] = _pack(t[:, 0:32])
        ob[...] = _pack(t[:, 32:64])


def _make_update_tc(with_pq):
    def body(h_ref, s0, s1, c0, c1, mw2, mb2, uw1h, uw1a, ub1, uw2, ub2,
             *rest):
        if with_pq:
            wj0, wj1, wi0, wi1, h_out, p0, p1, q0, q1 = rest
        else:
            h_out, = rest
        hb = h_ref[...]
        s = jnp.concatenate([_unpack(s0[...]), _unpack(s1[...])], axis=1)
        cnt_raw = c0[...][:, 0:1] + c1[...][:, 0:1]
        cnt = jnp.maximum(cnt_raw, 1.0)
        has_edges = jnp.minimum(cnt_raw, 1.0)
        aggr = _dot(s, mw2[...]) / cnt + has_edges * mb2[...]
        z = jnp.maximum(_dot(hb, uw1h[...]) + _dot(aggr, uw1a[...])
                        + ub1[...], 0.0)
        hn = hb + _dot(z, uw2[...]) + ub2[...]
        h_out[...] = hn
        if with_pq:
            p0[...] = _pack(_dot(hn, wj0[...]))
            p1[...] = _pack(_dot(hn, wj1[...]))
            q0[...] = _pack(_dot(hn, wi0[...]))
            q1[...] = _pack(_dot(hn, wi1[...]))
    return body


def _head_tc(h_ref, w1, b1, w2p, b2p, y_out):
    z = jnp.maximum(_dot(h_ref[...], w1[...]) + b1[...], 0.0)
    y_out[...] = _dot(z, w2p[...]) + b2p[...]


def _full(shape):
    return pl.BlockSpec(shape, lambda i: (0,) * len(shape))


def _rows(width):
    return pl.BlockSpec((BN, width), lambda i: (i, 0))


def _erows(width):
    return pl.BlockSpec((BE, width), lambda i: (i, 0))


def kernel(x, edge_index, edge_attr, node_W1, node_b1, node_W2, node_b2,
           edge_W1, edge_b1, edge_W2, edge_b2,
           msg_W1, msg_b1, msg_W2, msg_b2,
           upd_W1, upd_b1, upd_W2, upd_b2,
           head_W1, head_b1, head_W2, head_b2):
    x_p = jnp.zeros((NPAD, 8), _f32).at[:N, :5].set(x)
    src = edge_index[0]
    dst = edge_index[1]


    nW1p = jnp.zeros((8, H), _f32).at[:5].set(node_W1)
    r1 = lambda b: b.reshape(1, -1)

    mWj = msg_W1[:, 0:H, :]
    mWi = msg_W1[:, H:2 * H, :]
    mWe = msg_W1[:, 2 * H:3 * H, :]

    pq_pack_specs = [pl.BlockSpec((BN // 4, 128), lambda i: (i, 0))] * 4
    pq_pack_shapes = [jax.ShapeDtypeStruct((NPAD // 4, 128), _f32)] * 4

    node_call = pl.pallas_call(
        _node_tc,
        grid=(GRID_N,),
        in_specs=[_rows(8), _full((8, H)), _full((1, H)), _full((H, H)),
                  _full((1, H)), _full((H, 32)), _full((H, 32)),
                  _full((H, 32)), _full((H, 32))],
        out_specs=[_rows(H)] + pq_pack_specs,
        out_shape=[jax.ShapeDtypeStruct((NPAD, H), _f32)] + pq_pack_shapes,
    )
    h, P0, P1, Q0, Q1 = node_call(
        x_p, nW1p, r1(node_b1), node_W2, r1(node_b2),
        mWj[0][:, 0:32], mWj[0][:, 32:64], mWi[0][:, 0:32], mWi[0][:, 32:64])

    edgefeat_call = pl.pallas_call(
        _edgefeat_tc,
        grid=(GRID_E,),
        in_specs=[_erows(7), _full((7, H)), _full((1, H)), _full((H, H)),
                  _full((1, H))] + [_full((H, H))] * 3 + [_full((1, H))] * 3,
        out_specs=[pl.BlockSpec((BE // 4, 128), lambda i: (i, 0))] * 6,
        out_shape=[jax.ShapeDtypeStruct((E // 4, 128), _f32)] * 6,
    )
    ee = edgefeat_call(edge_attr, edge_W1, r1(edge_b1), edge_W2, r1(edge_b2),
                       mWe[0], mWe[1], mWe[2],
                       r1(msg_b1[0]), r1(msg_b1[1]), r1(msg_b1[2]))

    c0, c1 = _sc_count(dst)

    s_pack_spec = pl.BlockSpec((BN // 4, 128), lambda i: (i, 0))
    upd_in_specs = [_rows(H), s_pack_spec, s_pack_spec, _rows(16), _rows(16),
                    _full((H, H)), _full((1, H)), _full((H, H)),
                    _full((H, H)), _full((1, H)), _full((H, H)),
                    _full((1, H))]
    upd_pq_call = pl.pallas_call(
        _make_update_tc(True),
        grid=(GRID_N,),
        in_specs=upd_in_specs + [_full((H, 32))] * 4,
        out_specs=[_rows(H)] + pq_pack_specs,
        out_shape=[jax.ShapeDtypeStruct((NPAD, H), _f32)] + pq_pack_shapes,
    )
    upd_call = pl.pallas_call(
        _make_update_tc(False),
        grid=(GRID_N,),
        in_specs=upd_in_specs,
        out_specs=[_rows(H)],
        out_shape=[jax.ShapeDtypeStruct((NPAD, H), _f32)],
    )

    for l in range(3):
        s0, s1 = _sc_edge_pass(src, dst,
                               jnp.reshape(P0, (NPAD, 32)),
                               jnp.reshape(P1, (NPAD, 32)),
                               jnp.reshape(Q0, (NPAD, 32)),
                               jnp.reshape(Q1, (NPAD, 32)),
                               ee[2 * l], ee[2 * l + 1])
        common = (h, jnp.reshape(s0, (NPAD // 4, 128)),
                  jnp.reshape(s1, (NPAD // 4, 128)),
                  c0, c1, msg_W2[l], r1(msg_b2[l]),
                  upd_W1[l][0:H], upd_W1[l][H:2 * H], r1(upd_b1[l]),
                  upd_W2[l], r1(upd_b2[l]))
        if l < 2:
            h, P0, P1, Q0, Q1 = upd_pq_call(
                *common,
                mWj[l + 1][:, 0:32], mWj[l + 1][:, 32:64],
                mWi[l + 1][:, 0:32], mWi[l + 1][:, 32:64])
        else:
            h, = upd_call(*common)

    hW2p = jnp.zeros((H, 128), _f32).at[:, 0:2].set(head_W2)
    hb2p = jnp.zeros((1, 128), _f32).at[0, 0:2].set(head_b2)
    head_call = pl.pallas_call(
        _head_tc,
        grid=(1,),
        in_specs=[pl.BlockSpec((8, H), lambda i: (0, 0)), _full((H, H)),
                  _full((1, H)), _full((H, 128)), _full((1, 128))],
        out_specs=pl.BlockSpec((8, 128), lambda i: (0, 0)),
        out_shape=jax.ShapeDtypeStruct((8, 128), _f32),
    )
    y = head_call(h, head_W1, r1(head_b1), hW2p, hb2p)
    return y[0:1, 0:2]
